# Initial kernel scaffold; baseline (speedup 1.0000x reference)
import jax
import jax.numpy as jnp
from jax import lax
from jax.experimental import pallas as pl
from jax.experimental.pallas import tpu as pltpu
from jax.experimental.pallas import tpu_sc as plsc

_N = 10000
_NP = 10240
_E = 320000
_G = 64
_F = 128
_NW = 32
_CH = 128
_NCH = 79
_EP = _NW * _NCH * _CH
_DUMP = 16
_AR = _NP + _DUMP
_HI = jax.lax.Precision.HIGHEST


def _lrelu(x):
    return jnp.where(x >= 0, x, 0.2 * x)


def _edge_body(h_hbm, ev_hbm, m_hbm, src_hbm, dst_hbm, p_out, den_out,
               ev_v, m_v, si_v, di_v, rows_v, pv_v, acc_sh, den_sh, sem):
    cid = lax.axis_index("c")
    sid = lax.axis_index("s")
    wid = cid * 16 + sid
    base = sid * (_NP // 16)

    z16 = jnp.zeros((16,), jnp.float32)

    @pl.loop(0, _CH)
    def _(r):
        for c in range(8):
            rows_v[r, pl.ds(c * 16, 16)] = z16

    @pl.loop(0, 8)
    def _(j):
        pv_v[pl.ds(j * 16, 16)] = z16

    for k in range(_NP // 16 // _CH):
        pltpu.sync_copy(rows_v, acc_sh.at[pl.ds(base + k * _CH, _CH)])
        pltpu.sync_copy(pv_v, den_sh.at[pl.ds(base + k * _CH, _CH)])

    @pl.when(sid == 0)
    def _():
        pltpu.sync_copy(rows_v.at[pl.ds(0, _DUMP)], acc_sh.at[pl.ds(_NP, _DUMP)])
        pltpu.sync_copy(pv_v.at[pl.ds(0, _DUMP)], den_sh.at[pl.ds(_NP, _DUMP)])

    pltpu.sync_copy(ev_hbm, ev_v)
    pltpu.sync_copy(m_hbm.at[0, pl.ds(0, 16)], m_v)
    pltpu.sync_copy(src_hbm.at[wid], si_v)
    pltpu.sync_copy(dst_hbm.at[wid], di_v)
    plsc.subcore_barrier()

    m16 = m_v[...]
    zi16 = jnp.zeros((16,), jnp.int32)
    oi16 = jnp.ones((16,), jnp.int32)

    @pl.loop(0, _NCH)
    def _(ci):
        pltpu.async_copy(h_hbm.at[si_v.at[ci]], rows_v, sem).wait()

        @pl.loop(0, _CH // 16)
        def _(j):
            sl = pl.ds(j * 16, 16)
            iv = si_v[ci, sl]
            dv = jnp.minimum(di_v[ci, sl], _NP - 1)
            e = (plsc.load_gather(ev_v, [iv, zi16])
                 + plsc.load_gather(ev_v, [dv, oi16]))
            e = jnp.where(e >= 0.0, e, 0.2 * e)
            pv_v[sl] = jnp.exp(e - m16)

        @pl.loop(0, _CH)
        def _(r):
            pb = jnp.full((16,), pv_v[r])
            for c in range(8):
                sl = pl.ds(c * 16, 16)
                rows_v[r, sl] = rows_v[r, sl] * pb

        pltpu.sync_copy(rows_v, acc_sh.at[di_v.at[ci]], add=True)
        pltpu.sync_copy(pv_v, den_sh.at[di_v.at[ci]], add=True)

    plsc.subcore_barrier()
    out_sl = pl.ds(base, _NP // 16)
    pltpu.sync_copy(acc_sh.at[out_sl], p_out.at[cid, out_sl])
    pltpu.sync_copy(den_sh.at[out_sl], den_out.at[cid, out_sl])


def _sc_edge(h, ev, m, src3, dst3):
    kern = pl.kernel(
        _edge_body,
        out_type=[jax.ShapeDtypeStruct((2, _NP, _F), jnp.float32),
                  jax.ShapeDtypeStruct((2, _NP), jnp.float32)],
        mesh=plsc.VectorSubcoreMesh(core_axis_name="c", subcore_axis_name="s"),
        scratch_types=[
            pltpu.VMEM((_NP, 2), jnp.float32),
            pltpu.VMEM((16,), jnp.float32),
            pltpu.VMEM((_NCH, _CH), jnp.int32),
            pltpu.VMEM((_NCH, _CH), jnp.int32),
            pltpu.VMEM((_CH, _F), jnp.float32),
            pltpu.VMEM((_CH,), jnp.float32),
            pltpu.VMEM_SHARED((_AR, _F), jnp.float32),
            pltpu.VMEM_SHARED((_AR,), jnp.float32),
            pltpu.SemaphoreType.DMA,
        ],
    )
    return kern(h, ev, m, src3, dst3)


def _pre_body(x_ref, w_ref, av_ref, h_ref, ev_ref, m_ref):
    h = jnp.dot(x_ref[...], w_ref[...], precision=_HI)
    h_ref[...] = h
    ev = jnp.dot(h, av_ref[...], precision=_HI)
    ev_ref[...] = ev
    mx = jnp.max(ev, axis=0, keepdims=True)
    m = _lrelu(mx[0:1, 0:1] + mx[0:1, 1:2])
    m_ref[...] = jnp.broadcast_to(m, (8, 128))


def _tc_pre(x, w, av):
    return pl.pallas_call(
        _pre_body,
        out_shape=[jax.ShapeDtypeStruct((_NP, _F), jnp.float32),
                   jax.ShapeDtypeStruct((_NP, 2), jnp.float32),
                   jax.ShapeDtypeStruct((8, 128), jnp.float32)],
    )(x, w, av)


def _combine(p_ref, d_ref, ev_ref, m_ref, h_ref, b_ref):
    es = ev_ref[:, 0:1]
    ed = ev_ref[:, 1:2]
    m = m_ref[0:1, 0:1]
    ps = jnp.exp(_lrelu(es + ed) - m)
    h = h_ref[...]
    num = p_ref[0] + p_ref[1] + ps * h
    den = d_ref[0] + d_ref[1] + ps + 1e-16
    return num / den + b_ref[...]


def _mid_body(p_ref, d_ref, ev_ref, m_ref, h_ref, b_ref, w_ref, av_ref,
              h2_ref, ev2_ref, m2_ref):
    a = jnp.maximum(_combine(p_ref, d_ref, ev_ref, m_ref, h_ref, b_ref), 0.0)
    h2 = jnp.dot(a, w_ref[...], precision=_HI)
    h2_ref[...] = h2
    ev2 = jnp.dot(h2, av_ref[...], precision=_HI)
    ev2_ref[...] = ev2
    mx = jnp.max(ev2, axis=0, keepdims=True)
    m2 = _lrelu(mx[0:1, 0:1] + mx[0:1, 1:2])
    m2_ref[...] = jnp.broadcast_to(m2, (8, 128))


def _tc_mid(p, d, ev, m, h, b, w, av):
    return pl.pallas_call(
        _mid_body,
        out_shape=[jax.ShapeDtypeStruct((_NP, _F), jnp.float32),
                   jax.ShapeDtypeStruct((_NP, 2), jnp.float32),
                   jax.ShapeDtypeStruct((8, 128), jnp.float32)],
    )(p, d, ev, m, h, b, w, av)


def _fin_body(p_ref, d_ref, ev_ref, m_ref, h_ref, b_ref, batch_ref,
              wl_ref, bl_ref, out_ref):
    o = _combine(p_ref, d_ref, ev_ref, m_ref, h_ref, b_ref)
    bb = jnp.broadcast_to(batch_ref[...], (_G, _NP))
    gid = lax.broadcasted_iota(jnp.int32, (_G, _NP), 0)
    oh = (bb == gid).astype(jnp.float32)
    sums = jnp.dot(oh, o, precision=_HI)
    cnt = jnp.sum(oh, axis=1, keepdims=True)
    pooled = sums / jnp.maximum(cnt, 1.0)
    out_ref[...] = jnp.dot(pooled, wl_ref[...], precision=_HI) + bl_ref[...]


def _tc_fin(p, d, ev, m, h, b, batch2, wl, bl):
    return pl.pallas_call(
        _fin_body,
        out_shape=jax.ShapeDtypeStruct((_G, _F), jnp.float32),
    )(p, d, ev, m, h, b, batch2, wl, bl)


def kernel(x, edge_index, batch, W1, a_src1, a_dst1, b1, W2, a_src2, a_dst2,
           b2, W3, a_src3, a_dst3, b3, Wlin, blin):
    x_p = jnp.pad(x, ((0, _NP - _N), (0, 0)))
    src = edge_index[0]
    dst = edge_index[1]
    npad = _EP - _E
    src3 = jnp.concatenate(
        [src, jnp.zeros((npad,), jnp.int32)]).reshape(_NW, _NCH, _CH)
    dst3 = jnp.concatenate(
        [dst, _NP + (jnp.arange(npad, dtype=jnp.int32) % _DUMP)]
    ).reshape(_NW, _NCH, _CH)
    batch2 = jnp.concatenate(
        [batch, jnp.full((_NP - _N,), _G, jnp.int32)]).reshape(1, _NP)

    av1 = jnp.stack([a_src1, a_dst1], axis=1)
    av2 = jnp.stack([a_src2, a_dst2], axis=1)
    av3 = jnp.stack([a_src3, a_dst3], axis=1)

    h1, ev1, m1 = _tc_pre(x_p, W1, av1)
    P1, D1 = _sc_edge(h1, ev1, m1, src3, dst3)
    h2, ev2, m2 = _tc_mid(P1, D1.reshape(2, _NP, 1), ev1, m1, h1,
                          b1.reshape(1, _F), W2, av2)
    P2, D2 = _sc_edge(h2, ev2, m2, src3, dst3)
    h3, ev3, m3 = _tc_mid(P2, D2.reshape(2, _NP, 1), ev2, m2, h2,
                          b2.reshape(1, _F), W3, av3)
    P3, D3 = _sc_edge(h3, ev3, m3, src3, dst3)
    return _tc_fin(P3, D3.reshape(2, _NP, 1), ev3, m3, h3,
                   b3.reshape(1, _F), batch2, Wlin, blin.reshape(1, _F))


# speedup vs baseline: 20.2437x; 20.2437x over previous
import dataclasses

import jax
import jax.numpy as jnp
from jax import lax
from jax.experimental import pallas as pl
from jax.experimental.pallas import tpu as pltpu
from jax.experimental.pallas import tpu_sc as plsc

_N = 10000
_NP = 10240
_E = 320000
_G = 64
_F = 128
_NW = 32
_CH = 128
_NCH = 79
_EP = _NW * _NCH * _CH
_DUMP = 16
_AR = _NP + _DUMP
_HI = jax.lax.Precision.HIGHEST


def _lrelu(x):
    return jnp.where(x >= 0, x, 0.2 * x)


def _edge_body(h_hbm, ev_hbm, m_hbm, src_hbm, dst_hbm, p_out, den_out,
               ev_v, m_v, si_c, di_c, rows_v, pv_v, acc_sh, den_sh, sem):
    cid = lax.axis_index("c")
    sid = lax.axis_index("s")
    wid = cid * 16 + sid
    base = sid * (_NP // 16)

    z16 = jnp.zeros((16,), jnp.float32)

    @pl.loop(0, _CH)
    def _(r):
        for c in range(8):
            rows_v[r, pl.ds(c * 16, 16)] = z16

    @pl.loop(0, 8)
    def _(j):
        pv_v[pl.ds(j * 16, 16)] = z16

    for k in range(_NP // 16 // _CH):
        pltpu.sync_copy(rows_v, acc_sh.at[pl.ds(base + k * _CH, _CH)])
        pltpu.sync_copy(pv_v, den_sh.at[pl.ds(base + k * _CH, _CH)])

    @pl.when(sid == 0)
    def _():
        pltpu.sync_copy(rows_v.at[pl.ds(0, _DUMP)], acc_sh.at[pl.ds(_NP, _DUMP)])
        pltpu.sync_copy(pv_v.at[pl.ds(0, _DUMP)], den_sh.at[pl.ds(_NP, _DUMP)])

    pltpu.sync_copy(ev_hbm, ev_v)
    pltpu.sync_copy(m_hbm.at[0, pl.ds(0, 16)], m_v)
    plsc.subcore_barrier()

    m16 = m_v[...]
    one16 = jnp.ones((16,), jnp.int32)

    @pl.loop(0, _NCH)
    def _(ci):
        pltpu.sync_copy(src_hbm.at[wid, ci], si_c.at[0])
        pltpu.sync_copy(dst_hbm.at[wid, ci], di_c.at[0])
        pltpu.async_copy(h_hbm.at[si_c.at[0]], rows_v, sem).wait()

        @pl.loop(0, _CH // 16)
        def _(j):
            sl = pl.ds(j * 16, 16)
            iv = si_c[0, sl]
            dv = jnp.minimum(di_c[0, sl], _NP - 1)
            e = (plsc.load_gather(ev_v, [iv + iv])
                 + plsc.load_gather(ev_v, [dv + dv + one16]))
            e = jnp.where(e >= 0.0, e, 0.2 * e)
            pv_v[sl] = jnp.exp(e - m16)

        @pl.loop(0, _CH)
        def _(r):
            pb = plsc.load_gather(pv_v, [jnp.full((16,), r, jnp.int32)])
            for c in range(8):
                sl = pl.ds(c * 16, 16)
                rows_v[r, sl] = rows_v[r, sl] * pb

        pltpu.sync_copy(rows_v, acc_sh.at[di_c.at[0]], add=True)
        pltpu.sync_copy(pv_v, den_sh.at[di_c.at[0]], add=True)

    plsc.subcore_barrier()
    out_sl = pl.ds(base, _NP // 16)
    pltpu.sync_copy(acc_sh.at[out_sl], p_out.at[cid, out_sl])
    pltpu.sync_copy(den_sh.at[out_sl], den_out.at[cid, out_sl])


_SC_PARAMS = pltpu.CompilerParams()
if "needs_layout_passes" in pltpu.CompilerParams.__dataclass_fields__:
    _SC_PARAMS = dataclasses.replace(_SC_PARAMS, needs_layout_passes=False)


def _sc_edge(h, ev, m, src3, dst3):
    kern = pl.kernel(
        _edge_body,
        compiler_params=_SC_PARAMS,
        out_type=[jax.ShapeDtypeStruct((2, _NP, _F), jnp.float32),
                  jax.ShapeDtypeStruct((2, _NP), jnp.float32)],
        mesh=plsc.VectorSubcoreMesh(core_axis_name="c", subcore_axis_name="s"),
        scratch_types=[
            pltpu.VMEM((2 * _NP,), jnp.float32),
            pltpu.VMEM((16,), jnp.float32),
            pltpu.VMEM((1, _CH), jnp.int32),
            pltpu.VMEM((1, _CH), jnp.int32),
            pltpu.VMEM((_CH, _F), jnp.float32),
            pltpu.VMEM((_CH,), jnp.float32),
            pltpu.VMEM_SHARED((_AR, _F), jnp.float32),
            pltpu.VMEM_SHARED((_AR,), jnp.float32),
            pltpu.SemaphoreType.DMA,
        ],
    )
    return kern(h, ev, m, src3, dst3)


def _pre_body(x_ref, w_ref, av_ref, h_ref, ev_ref, m_ref):
    h = jnp.dot(x_ref[...], w_ref[...], precision=_HI)
    h_ref[...] = h
    ev = jnp.dot(h, av_ref[...], precision=_HI)
    ev_ref[...] = ev
    mx = jnp.max(ev, axis=0, keepdims=True)
    m = _lrelu(mx[0:1, 0:1] + mx[0:1, 1:2])
    m_ref[...] = jnp.broadcast_to(m, (8, 128))


def _tc_pre(x, w, av):
    return pl.pallas_call(
        _pre_body,
        out_shape=[jax.ShapeDtypeStruct((_NP, _F), jnp.float32),
                   jax.ShapeDtypeStruct((_NP, 2), jnp.float32),
                   jax.ShapeDtypeStruct((8, 128), jnp.float32)],
    )(x, w, av)


def _combine(p_ref, d_ref, ev_ref, m_ref, h_ref, b_ref):
    es = ev_ref[:, 0:1]
    ed = ev_ref[:, 1:2]
    m = m_ref[0:1, 0:1]
    ps = jnp.exp(_lrelu(es + ed) - m)
    h = h_ref[...]
    num = p_ref[0] + p_ref[1] + ps * h
    den = d_ref[0] + d_ref[1] + ps + 1e-16
    return num / den + b_ref[...]


def _mid_body(p_ref, d_ref, ev_ref, m_ref, h_ref, b_ref, w_ref, av_ref,
              h2_ref, ev2_ref, m2_ref):
    a = jnp.maximum(_combine(p_ref, d_ref, ev_ref, m_ref, h_ref, b_ref), 0.0)
    h2 = jnp.dot(a, w_ref[...], precision=_HI)
    h2_ref[...] = h2
    ev2 = jnp.dot(h2, av_ref[...], precision=_HI)
    ev2_ref[...] = ev2
    mx = jnp.max(ev2, axis=0, keepdims=True)
    m2 = _lrelu(mx[0:1, 0:1] + mx[0:1, 1:2])
    m2_ref[...] = jnp.broadcast_to(m2, (8, 128))


def _tc_mid(p, d, ev, m, h, b, w, av):
    return pl.pallas_call(
        _mid_body,
        out_shape=[jax.ShapeDtypeStruct((_NP, _F), jnp.float32),
                   jax.ShapeDtypeStruct((_NP, 2), jnp.float32),
                   jax.ShapeDtypeStruct((8, 128), jnp.float32)],
    )(p, d, ev, m, h, b, w, av)


def _fin_body(p_ref, d_ref, ev_ref, m_ref, h_ref, b_ref, batch_ref,
              wl_ref, bl_ref, out_ref):
    o = _combine(p_ref, d_ref, ev_ref, m_ref, h_ref, b_ref)
    bb = jnp.broadcast_to(batch_ref[...], (_G, _NP))
    gid = lax.broadcasted_iota(jnp.int32, (_G, _NP), 0)
    oh = (bb == gid).astype(jnp.float32)
    sums = jnp.dot(oh, o, precision=_HI)
    cnt = jnp.sum(oh, axis=1, keepdims=True)
    pooled = sums / jnp.maximum(cnt, 1.0)
    out_ref[...] = jnp.dot(pooled, wl_ref[...], precision=_HI) + bl_ref[...]


def _tc_fin(p, d, ev, m, h, b, batch2, wl, bl):
    return pl.pallas_call(
        _fin_body,
        out_shape=jax.ShapeDtypeStruct((_G, _F), jnp.float32),
    )(p, d, ev, m, h, b, batch2, wl, bl)


def kernel(x, edge_index, batch, W1, a_src1, a_dst1, b1, W2, a_src2, a_dst2,
           b2, W3, a_src3, a_dst3, b3, Wlin, blin):
    x_p = jnp.pad(x, ((0, _NP - _N), (0, 0)))
    src = edge_index[0]
    dst = edge_index[1]
    npad = _EP - _E
    src3 = jnp.concatenate(
        [src, jnp.zeros((npad,), jnp.int32)]).reshape(_NW, _NCH, _CH)
    dst3 = jnp.concatenate(
        [dst, _NP + (jnp.arange(npad, dtype=jnp.int32) % _DUMP)]
    ).reshape(_NW, _NCH, _CH)
    batch2 = jnp.concatenate(
        [batch, jnp.full((_NP - _N,), _G, jnp.int32)]).reshape(1, _NP)

    av1 = jnp.stack([a_src1, a_dst1], axis=1)
    av2 = jnp.stack([a_src2, a_dst2], axis=1)
    av3 = jnp.stack([a_src3, a_dst3], axis=1)

    h1, ev1, m1 = _tc_pre(x_p, W1, av1)
    P1, D1 = _sc_edge(h1, ev1.reshape(2 * _NP), m1, src3, dst3)
    h2, ev2, m2 = _tc_mid(P1, D1.reshape(2, _NP, 1), ev1, m1, h1,
                          b1.reshape(1, _F), W2, av2)
    P2, D2 = _sc_edge(h2, ev2.reshape(2 * _NP), m2, src3, dst3)
    h3, ev3, m3 = _tc_mid(P2, D2.reshape(2, _NP, 1), ev2, m2, h2,
                          b2.reshape(1, _F), W3, av3)
    P3, D3 = _sc_edge(h3, ev3.reshape(2 * _NP), m3, src3, dst3)
    return _tc_fin(P3, D3.reshape(2, _NP, 1), ev3, m3, h3,
                   b3.reshape(1, _F), batch2, Wlin, blin.reshape(1, _F))


# speedup vs baseline: 22.9333x; 1.1329x over previous
import dataclasses

import jax
import jax.numpy as jnp
from jax import lax
from jax.experimental import pallas as pl
from jax.experimental.pallas import tpu as pltpu
from jax.experimental.pallas import tpu_sc as plsc

_N = 10000
_NP = 10240
_E = 320000
_G = 64
_F = 128
_NW = 32
_CH = 64
_NCH = 158
_EP = _NW * _NCH * _CH
_DUMP = 16
_AR = _NP + _DUMP
_HI = jax.lax.Precision.HIGHEST


def _lrelu(x):
    return jnp.where(x >= 0, x, 0.2 * x)


def _edge_body(h_hbm, ev_hbm, m_hbm, pk_hbm, p_out, den_out,
               ev_v, m_v, pk_c, si_c, di_c, rows_v, pv_v, acc_sh, den_sh,
               si0, si1, sg0, sg1, ss0, ss1, sd0, sd1):
    cid = lax.axis_index("c")
    sid = lax.axis_index("s")
    wid = cid * 16 + sid
    base = sid * (_NP // 16)
    sem_i = (si0, si1)
    sem_g = (sg0, sg1)
    sem_s = (ss0, ss1)
    sem_d = (sd0, sd1)

    z16 = jnp.zeros((16,), jnp.float32)

    @pl.loop(0, _CH)
    def _(r):
        for b in range(2):
            for c in range(8):
                rows_v[b, r, pl.ds(c * 16, 16)] = z16

    for b in range(2):
        @pl.loop(0, _CH // 16)
        def _(j):
            pv_v[b, pl.ds(j * 16, 16)] = z16

    for k in range(_NP // 16 // _CH):
        bb = k % 2
        pltpu.sync_copy(rows_v.at[bb], acc_sh.at[pl.ds(base + k * _CH, _CH)])
        pltpu.sync_copy(pv_v.at[bb], den_sh.at[pl.ds(base + k * _CH, _CH)])

    @pl.when(sid == 0)
    def _():
        pltpu.sync_copy(rows_v.at[0, pl.ds(0, _DUMP)],
                        acc_sh.at[pl.ds(_NP, _DUMP)])
        pltpu.sync_copy(pv_v.at[0, pl.ds(0, _DUMP)],
                        den_sh.at[pl.ds(_NP, _DUMP)])

    pltpu.sync_copy(ev_hbm, ev_v)
    pltpu.sync_copy(m_hbm.at[0, pl.ds(0, 16)], m_v)
    plsc.subcore_barrier()

    m16 = m_v[...]
    one16 = jnp.ones((16,), jnp.int32)
    msk = jnp.full((16,), 0xFFFF, jnp.int32)

    for b in range(2):
        pltpu.async_copy(pk_hbm.at[wid, b], pk_c.at[b], sem_i[b])

    @pl.loop(0, _NCH // 2)
    def _(it):
        for b in range(2):
            ci = 2 * it + b
            pltpu.make_async_copy(pk_hbm.at[wid, ci], pk_c.at[b],
                                  sem_i[b]).wait()

            @pl.when(it > 0)
            def _():
                pltpu.make_async_copy(
                    rows_v.at[b], acc_sh.at[di_c.at[b]], sem_s[b]).wait()
                pltpu.make_async_copy(
                    pv_v.at[b], den_sh.at[di_c.at[b]], sem_d[b]).wait()

            @pl.loop(0, _CH // 16)
            def _(j):
                sl = pl.ds(j * 16, 16)
                pk16 = pk_c[b, sl]
                si_c[b, sl] = pk16 & msk
                di_c[b, sl] = jnp.right_shift(pk16, 16)

            pltpu.async_copy(h_hbm.at[si_c.at[b]], rows_v.at[b], sem_g[b])

            @pl.when(it < _NCH // 2 - 1)
            def _():
                pltpu.async_copy(pk_hbm.at[wid, ci + 2], pk_c.at[b],
                                 sem_i[b])

            @pl.loop(0, _CH // 16)
            def _(j):
                sl = pl.ds(j * 16, 16)
                iv = si_c[b, sl]
                dv = jnp.minimum(di_c[b, sl], _NP - 1)
                e = (plsc.load_gather(ev_v, [iv + iv])
                     + plsc.load_gather(ev_v, [dv + dv + one16]))
                e = jnp.where(e >= 0.0, e, 0.2 * e)
                pv_v[b, sl] = jnp.exp(e - m16)

            pltpu.make_async_copy(h_hbm.at[si_c.at[b]], rows_v.at[b],
                                  sem_g[b]).wait()

            b16 = jnp.full((16,), b, jnp.int32)

            @pl.loop(0, _CH)
            def _(r):
                pb = plsc.load_gather(pv_v, [b16, jnp.full((16,), r,
                                                           jnp.int32)])
                for c in range(8):
                    sl = pl.ds(c * 16, 16)
                    rows_v[b, r, sl] = rows_v[b, r, sl] * pb

            pltpu.async_copy(rows_v.at[b], acc_sh.at[di_c.at[b]],
                             sem_s[b], add=True)
            pltpu.async_copy(pv_v.at[b], den_sh.at[di_c.at[b]],
                             sem_d[b], add=True)

    for b in range(2):
        pltpu.make_async_copy(rows_v.at[b], acc_sh.at[di_c.at[b]],
                              sem_s[b]).wait()
        pltpu.make_async_copy(pv_v.at[b], den_sh.at[di_c.at[b]],
                              sem_d[b]).wait()

    plsc.subcore_barrier()
    out_sl = pl.ds(base, _NP // 16)
    pltpu.sync_copy(acc_sh.at[out_sl], p_out.at[cid, out_sl])
    pltpu.sync_copy(den_sh.at[out_sl], den_out.at[cid, out_sl])


_SC_PARAMS = pltpu.CompilerParams()
if "needs_layout_passes" in pltpu.CompilerParams.__dataclass_fields__:
    _SC_PARAMS = dataclasses.replace(_SC_PARAMS, needs_layout_passes=False)


def _sc_edge(h, ev, m, pk3):
    kern = pl.kernel(
        _edge_body,
        compiler_params=_SC_PARAMS,
        out_type=[jax.ShapeDtypeStruct((2, _NP, _F), jnp.float32),
                  jax.ShapeDtypeStruct((2, _NP), jnp.float32)],
        mesh=plsc.VectorSubcoreMesh(core_axis_name="c", subcore_axis_name="s"),
        scratch_types=[
            pltpu.VMEM((2 * _NP,), jnp.float32),
            pltpu.VMEM((16,), jnp.float32),
            pltpu.VMEM((2, _CH), jnp.int32),
            pltpu.VMEM((2, _CH), jnp.int32),
            pltpu.VMEM((2, _CH), jnp.int32),
            pltpu.VMEM((2, _CH, _F), jnp.float32),
            pltpu.VMEM((2, _CH), jnp.float32),
            pltpu.VMEM_SHARED((_AR, _F), jnp.float32),
            pltpu.VMEM_SHARED((_AR,), jnp.float32),
        ] + [pltpu.SemaphoreType.DMA] * 8,
    )
    return kern(h, ev, m, pk3)


def _pre_body(x_ref, w_ref, av_ref, h_ref, ev_ref, m_ref):
    h = jnp.dot(x_ref[...], w_ref[...], precision=_HI)
    h_ref[...] = h
    ev = jnp.dot(h, av_ref[...], precision=_HI)
    ev_ref[...] = ev
    mx = jnp.max(ev, axis=0, keepdims=True)
    m = _lrelu(mx[0:1, 0:1] + mx[0:1, 1:2])
    m_ref[...] = jnp.broadcast_to(m, (8, 128))


def _tc_pre(x, w, av):
    return pl.pallas_call(
        _pre_body,
        out_shape=[jax.ShapeDtypeStruct((_NP, _F), jnp.float32),
                   jax.ShapeDtypeStruct((_NP, 2), jnp.float32),
                   jax.ShapeDtypeStruct((8, 128), jnp.float32)],
    )(x, w, av)


def _combine(p_ref, d_ref, ev_ref, m_ref, h_ref, b_ref):
    es = ev_ref[:, 0:1]
    ed = ev_ref[:, 1:2]
    m = m_ref[0:1, 0:1]
    ps = jnp.exp(_lrelu(es + ed) - m)
    h = h_ref[...]
    num = p_ref[0] + p_ref[1] + ps * h
    den = d_ref[0] + d_ref[1] + ps + 1e-16
    return num / den + b_ref[...]


def _mid_body(p_ref, d_ref, ev_ref, m_ref, h_ref, b_ref, w_ref, av_ref,
              h2_ref, ev2_ref, m2_ref):
    a = jnp.maximum(_combine(p_ref, d_ref, ev_ref, m_ref, h_ref, b_ref), 0.0)
    h2 = jnp.dot(a, w_ref[...], precision=_HI)
    h2_ref[...] = h2
    ev2 = jnp.dot(h2, av_ref[...], precision=_HI)
    ev2_ref[...] = ev2
    mx = jnp.max(ev2, axis=0, keepdims=True)
    m2 = _lrelu(mx[0:1, 0:1] + mx[0:1, 1:2])
    m2_ref[...] = jnp.broadcast_to(m2, (8, 128))


def _tc_mid(p, d, ev, m, h, b, w, av):
    return pl.pallas_call(
        _mid_body,
        out_shape=[jax.ShapeDtypeStruct((_NP, _F), jnp.float32),
                   jax.ShapeDtypeStruct((_NP, 2), jnp.float32),
                   jax.ShapeDtypeStruct((8, 128), jnp.float32)],
    )(p, d, ev, m, h, b, w, av)


def _fin_body(p_ref, d_ref, ev_ref, m_ref, h_ref, b_ref, batch_ref,
              wl_ref, bl_ref, out_ref):
    o = _combine(p_ref, d_ref, ev_ref, m_ref, h_ref, b_ref)
    bb = jnp.broadcast_to(batch_ref[...], (_G, _NP))
    gid = lax.broadcasted_iota(jnp.int32, (_G, _NP), 0)
    oh = (bb == gid).astype(jnp.float32)
    sums = jnp.dot(oh, o, precision=_HI)
    cnt = jnp.sum(oh, axis=1, keepdims=True)
    pooled = sums / jnp.maximum(cnt, 1.0)
    out_ref[...] = jnp.dot(pooled, wl_ref[...], precision=_HI) + bl_ref[...]


def _tc_fin(p, d, ev, m, h, b, batch2, wl, bl):
    return pl.pallas_call(
        _fin_body,
        out_shape=jax.ShapeDtypeStruct((_G, _F), jnp.float32),
    )(p, d, ev, m, h, b, batch2, wl, bl)


def kernel(x, edge_index, batch, W1, a_src1, a_dst1, b1, W2, a_src2, a_dst2,
           b2, W3, a_src3, a_dst3, b3, Wlin, blin):
    x_p = jnp.pad(x, ((0, _NP - _N), (0, 0)))
    src = edge_index[0]
    dst = edge_index[1]
    npad = _EP - _E
    src_p = jnp.concatenate([src, jnp.zeros((npad,), jnp.int32)])
    dst_p = jnp.concatenate(
        [dst, _NP + (jnp.arange(npad, dtype=jnp.int32) % _DUMP)])
    pk3 = (src_p | (dst_p << 16)).reshape(_NW, _NCH, _CH)
    batch2 = jnp.concatenate(
        [batch, jnp.full((_NP - _N,), _G, jnp.int32)]).reshape(1, _NP)

    av1 = jnp.stack([a_src1, a_dst1], axis=1)
    av2 = jnp.stack([a_src2, a_dst2], axis=1)
    av3 = jnp.stack([a_src3, a_dst3], axis=1)

    h1, ev1, m1 = _tc_pre(x_p, W1, av1)
    P1, D1 = _sc_edge(h1, ev1.reshape(2 * _NP), m1, pk3)
    h2, ev2, m2 = _tc_mid(P1, D1.reshape(2, _NP, 1), ev1, m1, h1,
                          b1.reshape(1, _F), W2, av2)
    P2, D2 = _sc_edge(h2, ev2.reshape(2 * _NP), m2, pk3)
    h3, ev3, m3 = _tc_mid(P2, D2.reshape(2, _NP, 1), ev2, m2, h2,
                          b2.reshape(1, _F), W3, av3)
    P3, D3 = _sc_edge(h3, ev3.reshape(2 * _NP), m3, pk3)
    return _tc_fin(P3, D3.reshape(2, _NP, 1), ev3, m3, h3,
                   b3.reshape(1, _F), batch2, Wlin, blin.reshape(1, _F))
